# baseline (speedup 1.0000x reference)
import functools

import jax
import jax.numpy as jnp
from jax import lax
from jax.experimental import pallas as pl
from jax.experimental.pallas import tpu as pltpu
from jax.experimental.pallas import tpu_sc as plsc

B = 256
K = 1024
F = 128
W = 4


def _vgather(vec, idx):
  dnums = lax.GatherDimensionNumbers(
      offset_dims=(), collapsed_slice_dims=(0,), start_index_map=(0,))
  return lax.gather(vec, idx[:, None], dnums, slice_sizes=(1,),
                    mode=lax.GatherScatterMode.PROMISE_IN_BOUNDS)


@jax.jit
def _sc_switch(query, km3, sv):
  info = plsc.get_sparse_core_info()
  NC, NS, L = info.num_cores, info.num_subcores, info.num_lanes
  NT = NC * NS
  QPT = B // NT
  RPS = K // NS
  NG = RPS // L

  mesh = plsc.VectorSubcoreMesh(core_axis_name="c", subcore_axis_name="s")

  @functools.partial(
      pl.kernel,
      out_type=jax.ShapeDtypeStruct((NT, QPT), jnp.float32),
      mesh=mesh,
      compiler_params=pltpu.CompilerParams(
          needs_layout_passes=False, use_tc_tiling_on_sc=False,
          skip_device_barrier=True),
      scratch_types=[
          pltpu.VMEM((QPT, F), jnp.float32),
          pltpu.VMEM((F, RPS), jnp.float32),
          pltpu.VMEM((2, W, RPS), jnp.int32),
          pltpu.VMEM_SHARED((NS, 2, W, RPS), jnp.int32),
          pltpu.VMEM((NS, 2, W, RPS), jnp.int32),
          pltpu.VMEM((2, K), jnp.float32),
          pltpu.VMEM((K,), jnp.int32),
          pltpu.VMEM((16,), jnp.float32),
          pltpu.SemaphoreType.DMA,
          pltpu.SemaphoreType.DMA,
          pltpu.SemaphoreType.DMA,
      ],
  )
  def body(q_hbm, km3_hbm, sv_hbm, out_hbm,
           qstage, kmstage, kcpl, kcp_sh, kcp, sv_v, code_v, outv,
           sem_q, sem_km, sem_sv):
    cid = lax.axis_index("c")
    sid = lax.axis_index("s")
    wid = cid * NS + sid

    cq = pltpu.async_copy(q_hbm.at[pl.ds(wid * QPT, QPT)], qstage, sem_q)
    ckm = pltpu.async_copy(km3_hbm.at[:, pl.ds(sid * RPS, RPS)], kmstage,
                           sem_km)
    csv = pltpu.async_copy(sv_hbm, sv_v, sem_sv)

    lanes = lax.iota(jnp.int32, L)
    pow2 = lax.shift_left(jnp.ones((L,), jnp.int32), lanes)
    ione = jnp.full((L,), 1, jnp.int32)

    ckm.wait()
    for w in range(W):
      def rbody(j, accs):
        kaccs, caccs = accs
        f = 32 * w + j
        sh = jnp.full((L,), j, jnp.int32)
        kaccs_n = []
        caccs_n = []
        for lg in range(NG):
          km = kmstage[f, pl.ds(lg * L, L)].astype(jnp.int32)
          kbit = km & ione
          cbit = (km >> 1) ^ ione
          kaccs_n.append(kaccs[lg] | lax.shift_left(kbit, sh))
          caccs_n.append(caccs[lg] | lax.shift_left(cbit, sh))
        return tuple(kaccs_n), tuple(caccs_n)
      zero = jnp.zeros((L,), jnp.int32)
      kaccs, caccs = lax.fori_loop(
          0, 32, rbody,
          (tuple(zero for _ in range(NG)), tuple(zero for _ in range(NG))),
          unroll=2)
      for lg in range(NG):
        kcpl[0, w, pl.ds(lg * L, L)] = kaccs[lg]
        kcpl[1, w, pl.ds(lg * L, L)] = caccs[lg]

    pltpu.sync_copy(kcpl, kcp_sh.at[sid])
    plsc.subcore_barrier()
    pltpu.sync_copy(kcp_sh, kcp)
    cq.wait()
    csv.wait()

    def code_body(s, _):
      for i in range(NG):
        base = s * RPS + i * L
        vbit = sv_v[1, pl.ds(base, L)].astype(jnp.int32)
        code_v[pl.ds(base, L)] = ((base + lanes) << 1) | vbit
      return 0
    lax.fori_loop(0, NS, code_body, 0)

    v0 = jnp.max(_vgather(sv_v[1, pl.ds(0, L)], jnp.zeros((L,), jnp.int32)))
    fzero = jnp.zeros((L,), jnp.float32)
    izero = jnp.zeros((L,), jnp.int32)
    big = jnp.full((L,), 2 * K, jnp.int32)
    neg1 = jnp.full((L,), -1, jnp.int32)

    NQ = 4

    def quad_body(bp, predv):
      b0 = NQ * bp
      qw = []
      for b in range(NQ):
        wl = []
        for w in range(W):
          lo = jnp.sum(
              qstage[b0 + b, pl.ds(32 * w, L)].astype(jnp.int32) * pow2)
          hi = jnp.sum(
              qstage[b0 + b, pl.ds(32 * w + L, L)].astype(jnp.int32) * pow2)
          wl.append(jnp.broadcast_to(lo | (hi << 16), (L,)))
        qw.append(wl)

      def scan(s, st):
        st = list(st)
        for i in range(NG):
          base = s * RPS + i * L
          kpw = [kcp[s, 0, w, pl.ds(i * L, L)] for w in range(W)]
          cpw = [kcp[s, 1, w, pl.ds(i * L, L)] for w in range(W)]
          sbits = plsc.bitcast(sv_v[0, pl.ds(base, L)], jnp.int32)
          codev = code_v[pl.ds(base, L)]

          for qb in range(NQ):
            bbits, bcode = st[2 * qb:2 * qb + 2]
            mism = ((qw[qb][0] ^ kpw[0]) & cpw[0])
            for w in range(1, W):
              mism = mism | ((qw[qb][w] ^ kpw[w]) & cpw[w])
            gt = (mism == izero) & (sbits > bbits)
            st[2 * qb:2 * qb + 2] = (jnp.where(gt, sbits, bbits),
                                     jnp.where(gt, codev, bcode))
        return tuple(st)

      st = lax.fori_loop(0, NS, scan, (neg1, big) * NQ)

      for i in range(NQ):
        bbits, bcode = st[2 * i:2 * i + 2]
        m = jnp.max(bbits)
        ksc = jnp.min(jnp.where(bbits == m, bcode, big))
        val = (ksc & 1).astype(jnp.float32)
        pred_b = jnp.where(m > 0, val, jnp.where(m == 0, v0, 0.0))
        predv = jnp.where(lanes == b0 + i, pred_b, predv)
      return predv

    predv = lax.fori_loop(0, QPT // NQ, quad_body, fzero)

    outv[...] = predv
    pltpu.sync_copy(outv.at[pl.ds(0, QPT)], out_hbm.at[wid])

  return body(query, km3, sv)


def kernel(query, label, keys, masks, values, scores):
  del label
  ns = 16
  km3 = (keys + 2.0 * masks).T
  sv = jnp.stack([scores, values])
  out = _sc_switch(query, km3, sv)
  return out.reshape(B)


# speedup vs baseline: 1.0224x; 1.0224x over previous
import functools

import jax
import jax.numpy as jnp
from jax import lax
from jax.experimental import pallas as pl
from jax.experimental.pallas import tpu as pltpu
from jax.experimental.pallas import tpu_sc as plsc

B = 256
K = 1024
F = 128
W = 4


def _vgather(vec, idx):
  dnums = lax.GatherDimensionNumbers(
      offset_dims=(), collapsed_slice_dims=(0,), start_index_map=(0,))
  return lax.gather(vec, idx[:, None], dnums, slice_sizes=(1,),
                    mode=lax.GatherScatterMode.PROMISE_IN_BOUNDS)


@jax.jit
def _sc_switch(query, km3, sv):
  info = plsc.get_sparse_core_info()
  NC, NS, L = info.num_cores, info.num_subcores, info.num_lanes
  NT = NC * NS
  QPT = B // NT
  RPS = K // NS
  NG = RPS // L

  mesh = plsc.VectorSubcoreMesh(core_axis_name="c", subcore_axis_name="s")

  @functools.partial(
      pl.kernel,
      out_type=jax.ShapeDtypeStruct((NT, QPT), jnp.float32),
      mesh=mesh,
      compiler_params=pltpu.CompilerParams(
          needs_layout_passes=False, use_tc_tiling_on_sc=False,
          skip_device_barrier=True),
      scratch_types=[
          pltpu.VMEM((QPT, F), jnp.float32),
          pltpu.VMEM((F, RPS), jnp.float32),
          pltpu.VMEM((2, W, RPS), jnp.int32),
          pltpu.VMEM_SHARED((NS, 2, W, RPS), jnp.int32),
          pltpu.VMEM((NS, 2, W, RPS), jnp.int32),
          pltpu.VMEM((2, K), jnp.float32),
          pltpu.VMEM((K,), jnp.int32),
          pltpu.VMEM((16,), jnp.float32),
          pltpu.SemaphoreType.DMA,
          pltpu.SemaphoreType.DMA,
          pltpu.SemaphoreType.DMA,
      ],
  )
  def body(q_hbm, km3_hbm, sv_hbm, out_hbm,
           qstage, kmstage, kcpl, kcp_sh, kcp, sv_v, code_v, outv,
           sem_q, sem_km, sem_sv):
    cid = lax.axis_index("c")
    sid = lax.axis_index("s")
    wid = cid * NS + sid

    cq = pltpu.async_copy(q_hbm.at[pl.ds(wid * QPT, QPT)], qstage, sem_q)
    ckm = pltpu.async_copy(km3_hbm.at[:, pl.ds(sid * RPS, RPS)], kmstage,
                           sem_km)
    csv = pltpu.async_copy(sv_hbm, sv_v, sem_sv)

    lanes = lax.iota(jnp.int32, L)
    pow2 = lax.shift_left(jnp.ones((L,), jnp.int32), lanes)
    ione = jnp.full((L,), 1, jnp.int32)

    ckm.wait()
    for w in range(W):
      def rbody(j, accs):
        kaccs, caccs = accs
        f = 32 * w + j
        sh = jnp.full((L,), j, jnp.int32)
        kaccs_n = []
        caccs_n = []
        for lg in range(NG):
          km = kmstage[f, pl.ds(lg * L, L)].astype(jnp.int32)
          kbit = km & ione
          cbit = (km >> 1) ^ ione
          kaccs_n.append(kaccs[lg] | lax.shift_left(kbit, sh))
          caccs_n.append(caccs[lg] | lax.shift_left(cbit, sh))
        return tuple(kaccs_n), tuple(caccs_n)
      zero = jnp.zeros((L,), jnp.int32)
      kaccs, caccs = lax.fori_loop(
          0, 32, rbody,
          (tuple(zero for _ in range(NG)), tuple(zero for _ in range(NG))),
          unroll=2)
      for lg in range(NG):
        kcpl[0, w, pl.ds(lg * L, L)] = kaccs[lg]
        kcpl[1, w, pl.ds(lg * L, L)] = caccs[lg]

    pltpu.sync_copy(kcpl, kcp_sh.at[sid])
    plsc.subcore_barrier()
    pltpu.sync_copy(kcp_sh, kcp)
    cq.wait()
    csv.wait()

    def code_body(s, _):
      for i in range(NG):
        base = s * RPS + i * L
        vbit = sv_v[1, pl.ds(base, L)].astype(jnp.int32)
        code_v[pl.ds(base, L)] = ((base + lanes) << 1) | vbit
      return 0
    lax.fori_loop(0, NS, code_body, 0)

    v0 = jnp.max(_vgather(sv_v[1, pl.ds(0, L)], jnp.zeros((L,), jnp.int32)))
    fzero = jnp.zeros((L,), jnp.float32)
    izero = jnp.zeros((L,), jnp.int32)
    big = jnp.full((L,), 2 * K, jnp.int32)
    neg1 = jnp.full((L,), -1, jnp.int32)

    NQ = 2

    def quad_body(bp, predv):
      b0 = NQ * bp
      qw = []
      for b in range(NQ):
        wl = []
        for w in range(W):
          lo = jnp.sum(
              qstage[b0 + b, pl.ds(32 * w, L)].astype(jnp.int32) * pow2)
          hi = jnp.sum(
              qstage[b0 + b, pl.ds(32 * w + L, L)].astype(jnp.int32) * pow2)
          wl.append(jnp.broadcast_to(lo | (hi << 16), (L,)))
        qw.append(wl)

      def scan(s, st):
        st = list(st)
        for i in range(NG):
          base = s * RPS + i * L
          kpw = [kcp[s, 0, w, pl.ds(i * L, L)] for w in range(W)]
          cpw = [kcp[s, 1, w, pl.ds(i * L, L)] for w in range(W)]
          sbits = plsc.bitcast(sv_v[0, pl.ds(base, L)], jnp.int32)
          codev = code_v[pl.ds(base, L)]

          for qb in range(NQ):
            bbits, bcode = st[2 * qb:2 * qb + 2]
            mism = ((qw[qb][0] ^ kpw[0]) & cpw[0])
            for w in range(1, W):
              mism = mism | ((qw[qb][w] ^ kpw[w]) & cpw[w])
            gt = (mism == izero) & (sbits > bbits)
            st[2 * qb:2 * qb + 2] = (jnp.where(gt, sbits, bbits),
                                     jnp.where(gt, codev, bcode))
        return tuple(st)

      st = lax.fori_loop(0, NS, scan, (neg1, big) * NQ)

      for i in range(NQ):
        bbits, bcode = st[2 * i:2 * i + 2]
        m = jnp.max(bbits)
        ksc = jnp.min(jnp.where(bbits == m, bcode, big))
        val = (ksc & 1).astype(jnp.float32)
        pred_b = jnp.where(m > 0, val, jnp.where(m == 0, v0, 0.0))
        predv = jnp.where(lanes == b0 + i, pred_b, predv)
      return predv

    predv = lax.fori_loop(0, QPT // NQ, quad_body, fzero)

    outv[...] = predv
    pltpu.sync_copy(outv.at[pl.ds(0, QPT)], out_hbm.at[wid])

  return body(query, km3, sv)


def kernel(query, label, keys, masks, values, scores):
  del label
  ns = 16
  km3 = (keys + 2.0 * masks).T
  sv = jnp.stack([scores, values])
  out = _sc_switch(query, km3, sv)
  return out.reshape(B)


# speedup vs baseline: 1.0224x; 1.0001x over previous
import functools

import jax
import jax.numpy as jnp
from jax import lax
from jax.experimental import pallas as pl
from jax.experimental.pallas import tpu as pltpu
from jax.experimental.pallas import tpu_sc as plsc

B = 256
K = 1024
F = 128
W = 4


def _vgather(vec, idx):
  dnums = lax.GatherDimensionNumbers(
      offset_dims=(), collapsed_slice_dims=(0,), start_index_map=(0,))
  return lax.gather(vec, idx[:, None], dnums, slice_sizes=(1,),
                    mode=lax.GatherScatterMode.PROMISE_IN_BOUNDS)


@jax.jit
def _sc_switch(query, km3, sv):
  info = plsc.get_sparse_core_info()
  NC, NS, L = info.num_cores, info.num_subcores, info.num_lanes
  NT = NC * NS
  QPT = B // NT
  RPS = K // NS
  NG = RPS // L

  mesh = plsc.VectorSubcoreMesh(core_axis_name="c", subcore_axis_name="s")

  @functools.partial(
      pl.kernel,
      out_type=jax.ShapeDtypeStruct((NT, QPT), jnp.float32),
      mesh=mesh,
      compiler_params=pltpu.CompilerParams(
          needs_layout_passes=False, use_tc_tiling_on_sc=False,
          skip_device_barrier=True),
      scratch_types=[
          pltpu.VMEM((QPT, F), jnp.float32),
          pltpu.VMEM((F, RPS), jnp.float32),
          pltpu.VMEM((2, W, RPS), jnp.int32),
          pltpu.VMEM_SHARED((NS, 2, W, RPS), jnp.int32),
          pltpu.VMEM((NS, 2, W, RPS), jnp.int32),
          pltpu.VMEM((2, K), jnp.float32),
          pltpu.VMEM((16,), jnp.float32),
          pltpu.SemaphoreType.DMA,
          pltpu.SemaphoreType.DMA,
          pltpu.SemaphoreType.DMA,
      ],
  )
  def body(q_hbm, km3_hbm, sv_hbm, out_hbm,
           qstage, kmstage, kcpl, kcp_sh, kcp, sv_v, outv,
           sem_q, sem_km, sem_sv):
    cid = lax.axis_index("c")
    sid = lax.axis_index("s")
    wid = cid * NS + sid

    cq = pltpu.async_copy(q_hbm.at[pl.ds(wid * QPT, QPT)], qstage, sem_q)
    ckm = pltpu.async_copy(km3_hbm.at[:, pl.ds(sid * RPS, RPS)], kmstage,
                           sem_km)
    csv = pltpu.async_copy(sv_hbm, sv_v, sem_sv)

    lanes = lax.iota(jnp.int32, L)
    pow2 = lax.shift_left(jnp.ones((L,), jnp.int32), lanes)
    ione = jnp.full((L,), 1, jnp.int32)

    ckm.wait()
    for w in range(W):
      def rbody(j, accs):
        kaccs, caccs = accs
        f = 32 * w + j
        sh = jnp.full((L,), j, jnp.int32)
        kaccs_n = []
        caccs_n = []
        for lg in range(NG):
          km = kmstage[f, pl.ds(lg * L, L)].astype(jnp.int32)
          kbit = km & ione
          cbit = (km >> 1) ^ ione
          kaccs_n.append(kaccs[lg] | lax.shift_left(kbit, sh))
          caccs_n.append(caccs[lg] | lax.shift_left(cbit, sh))
        return tuple(kaccs_n), tuple(caccs_n)
      zero = jnp.zeros((L,), jnp.int32)
      kaccs, caccs = lax.fori_loop(
          0, 32, rbody,
          (tuple(zero for _ in range(NG)), tuple(zero for _ in range(NG))),
          unroll=2)
      for lg in range(NG):
        kcpl[0, w, pl.ds(lg * L, L)] = kaccs[lg]
        kcpl[1, w, pl.ds(lg * L, L)] = caccs[lg]

    pltpu.sync_copy(kcpl, kcp_sh.at[sid])
    plsc.subcore_barrier()
    pltpu.sync_copy(kcp_sh, kcp)
    cq.wait()
    csv.wait()

    v0 = jnp.max(_vgather(sv_v[1, pl.ds(0, L)], jnp.zeros((L,), jnp.int32)))
    fzero = jnp.zeros((L,), jnp.float32)
    izero = jnp.zeros((L,), jnp.int32)
    big = jnp.full((L,), K, jnp.int32)
    fmask = jnp.zeros((L,), jnp.bool_)

    NQ = 2

    def quad_body(bp, predv):
      b0 = NQ * bp
      qw = []
      for b in range(NQ):
        wl = []
        for w in range(W):
          lo = jnp.sum(
              qstage[b0 + b, pl.ds(32 * w, L)].astype(jnp.int32) * pow2)
          hi = jnp.sum(
              qstage[b0 + b, pl.ds(32 * w + L, L)].astype(jnp.int32) * pow2)
          wl.append(jnp.broadcast_to(lo | (hi << 16), (L,)))
        qw.append(wl)

      def scan(s, st):
        st = list(st)
        for i in range(NG):
          base = s * RPS + i * L
          kpw = [kcp[s, 0, w, pl.ds(i * L, L)] for w in range(W)]
          cpw = [kcp[s, 1, w, pl.ds(i * L, L)] for w in range(W)]
          svec = sv_v[0, pl.ds(base, L)]
          vvec = sv_v[1, pl.ds(base, L)]
          idxv = base + lanes

          for qb in range(NQ):
            best, bidx, bval, anym = st[4 * qb:4 * qb + 4]
            mism = ((qw[qb][0] ^ kpw[0]) & cpw[0])
            for w in range(1, W):
              mism = mism | ((qw[qb][w] ^ kpw[w]) & cpw[w])
            match = mism == izero
            gt = match & (svec > best)
            st[4 * qb:4 * qb + 4] = (jnp.where(gt, svec, best),
                                     jnp.where(gt, idxv, bidx),
                                     jnp.where(gt, vvec, bval),
                                     anym | match)
        return tuple(st)

      st = lax.fori_loop(0, NS, scan,
                         (fzero, big, fzero, fmask) * NQ)

      for i in range(NQ):
        best, bidx, bval, anym = st[4 * i:4 * i + 4]
        m = jnp.max(best)
        win = best == m
        ks = jnp.min(jnp.where(win, bidx, big))
        winner = win & (bidx == ks)
        val = jnp.sum(jnp.where(winner, bval, fzero))
        anys = jnp.max(jnp.where(anym, 1, 0).astype(jnp.int32))
        pred_b = jnp.where(m > 0.0, val,
                           jnp.where(anys > 0, v0, 0.0))
        predv = jnp.where(lanes == b0 + i, pred_b, predv)
      return predv

    predv = lax.fori_loop(0, QPT // NQ, quad_body, fzero)

    outv[...] = predv
    pltpu.sync_copy(outv.at[pl.ds(0, QPT)], out_hbm.at[wid])

  return body(query, km3, sv)


def kernel(query, label, keys, masks, values, scores):
  del label
  ns = 16
  km3 = (keys + 2.0 * masks).T
  sv = jnp.stack([scores, values])
  out = _sc_switch(query, km3, sv)
  return out.reshape(B)


# speedup vs baseline: 1.0309x; 1.0082x over previous
import functools

import jax
import jax.numpy as jnp
from jax import lax
from jax.experimental import pallas as pl
from jax.experimental.pallas import tpu as pltpu
from jax.experimental.pallas import tpu_sc as plsc

B = 256
K = 1024
F = 128
W = 4


def _vgather(vec, idx):
  dnums = lax.GatherDimensionNumbers(
      offset_dims=(), collapsed_slice_dims=(0,), start_index_map=(0,))
  return lax.gather(vec, idx[:, None], dnums, slice_sizes=(1,),
                    mode=lax.GatherScatterMode.PROMISE_IN_BOUNDS)


@jax.jit
def _sc_switch(query, km3, sv):
  info = plsc.get_sparse_core_info()
  NC, NS, L = info.num_cores, info.num_subcores, info.num_lanes
  NT = NC * NS
  QPT = B // NT
  RPS = K // NS
  NG = RPS // L

  mesh = plsc.VectorSubcoreMesh(core_axis_name="c", subcore_axis_name="s")

  @functools.partial(
      pl.kernel,
      out_type=jax.ShapeDtypeStruct((NT, QPT), jnp.float32),
      mesh=mesh,
      compiler_params=pltpu.CompilerParams(
          needs_layout_passes=False, use_tc_tiling_on_sc=False,
          skip_device_barrier=True),
      scratch_types=[
          pltpu.VMEM((QPT, F), jnp.float32),
          pltpu.VMEM((F, RPS), jnp.float32),
          pltpu.VMEM((2, W, RPS), jnp.int32),
          pltpu.VMEM_SHARED((NS, 2, W, RPS), jnp.int32),
          pltpu.VMEM((NS, 2, W, RPS), jnp.int32),
          pltpu.VMEM((2, K), jnp.float32),
          pltpu.VMEM((16,), jnp.float32),
          pltpu.SemaphoreType.DMA,
          pltpu.SemaphoreType.DMA,
          pltpu.SemaphoreType.DMA,
      ],
  )
  def body(q_hbm, km3_hbm, sv_hbm, out_hbm,
           qstage, kmstage, kcpl, kcp_sh, kcp, sv_v, outv,
           sem_q, sem_km, sem_sv):
    cid = lax.axis_index("c")
    sid = lax.axis_index("s")
    wid = cid * NS + sid

    cq = pltpu.async_copy(q_hbm.at[pl.ds(wid * QPT, QPT)], qstage, sem_q)
    ckm = pltpu.async_copy(km3_hbm.at[:, pl.ds(sid * RPS, RPS)], kmstage,
                           sem_km)
    csv = pltpu.async_copy(sv_hbm, sv_v, sem_sv)

    lanes = lax.iota(jnp.int32, L)
    pow2 = lax.shift_left(jnp.ones((L,), jnp.int32), lanes)
    ione = jnp.full((L,), 1, jnp.int32)

    ckm.wait()
    def wbody(w, _):
      def rbody(j, accs):
        kaccs, caccs = accs
        f = 32 * w + j
        sh = jnp.full((L,), j, jnp.int32)
        kaccs_n = []
        caccs_n = []
        for lg in range(NG):
          km = kmstage[f, pl.ds(lg * L, L)].astype(jnp.int32)
          kbit = km & ione
          cbit = (km >> 1) ^ ione
          kaccs_n.append(kaccs[lg] | lax.shift_left(kbit, sh))
          caccs_n.append(caccs[lg] | lax.shift_left(cbit, sh))
        return tuple(kaccs_n), tuple(caccs_n)
      zero = jnp.zeros((L,), jnp.int32)
      kaccs, caccs = lax.fori_loop(
          0, 32, rbody,
          (tuple(zero for _ in range(NG)), tuple(zero for _ in range(NG))),
          unroll=2)
      for lg in range(NG):
        kcpl[0, w, pl.ds(lg * L, L)] = kaccs[lg]
        kcpl[1, w, pl.ds(lg * L, L)] = caccs[lg]
      return 0
    lax.fori_loop(0, W, wbody, 0)

    pltpu.sync_copy(kcpl, kcp_sh.at[sid])
    plsc.subcore_barrier()
    pltpu.sync_copy(kcp_sh, kcp)
    cq.wait()
    csv.wait()

    v0 = jnp.max(_vgather(sv_v[1, pl.ds(0, L)], jnp.zeros((L,), jnp.int32)))
    fzero = jnp.zeros((L,), jnp.float32)
    izero = jnp.zeros((L,), jnp.int32)
    big = jnp.full((L,), K, jnp.int32)
    fmask = jnp.zeros((L,), jnp.bool_)

    NQ = 2

    def quad_body(bp, predv):
      b0 = NQ * bp
      qw = []
      for b in range(NQ):
        wl = []
        for w in range(W):
          lo = jnp.sum(
              qstage[b0 + b, pl.ds(32 * w, L)].astype(jnp.int32) * pow2)
          hi = jnp.sum(
              qstage[b0 + b, pl.ds(32 * w + L, L)].astype(jnp.int32) * pow2)
          wl.append(jnp.broadcast_to(lo | (hi << 16), (L,)))
        qw.append(wl)

      def scan(s, st):
        st = list(st)
        for i in range(NG):
          base = s * RPS + i * L
          kpw = [kcp[s, 0, w, pl.ds(i * L, L)] for w in range(W)]
          cpw = [kcp[s, 1, w, pl.ds(i * L, L)] for w in range(W)]
          svec = sv_v[0, pl.ds(base, L)]
          vvec = sv_v[1, pl.ds(base, L)]
          idxv = base + lanes

          for qb in range(NQ):
            best, bidx, bval, anym = st[4 * qb:4 * qb + 4]
            mism = ((qw[qb][0] ^ kpw[0]) & cpw[0])
            for w in range(1, W):
              mism = mism | ((qw[qb][w] ^ kpw[w]) & cpw[w])
            match = mism == izero
            gt = match & (svec > best)
            st[4 * qb:4 * qb + 4] = (jnp.where(gt, svec, best),
                                     jnp.where(gt, idxv, bidx),
                                     jnp.where(gt, vvec, bval),
                                     anym | match)
        return tuple(st)

      st = lax.fori_loop(0, NS, scan,
                         (fzero, big, fzero, fmask) * NQ)

      for i in range(NQ):
        best, bidx, bval, anym = st[4 * i:4 * i + 4]
        m = jnp.max(best)
        win = best == m
        ks = jnp.min(jnp.where(win, bidx, big))
        winner = win & (bidx == ks)
        val = jnp.sum(jnp.where(winner, bval, fzero))
        anys = jnp.max(jnp.where(anym, 1, 0).astype(jnp.int32))
        pred_b = jnp.where(m > 0.0, val,
                           jnp.where(anys > 0, v0, 0.0))
        predv = jnp.where(lanes == b0 + i, pred_b, predv)
      return predv

    predv = lax.fori_loop(0, QPT // NQ, quad_body, fzero)

    outv[...] = predv
    pltpu.sync_copy(outv.at[pl.ds(0, QPT)], out_hbm.at[wid])

  return body(query, km3, sv)


def kernel(query, label, keys, masks, values, scores):
  del label
  ns = 16
  km3 = (keys + 2.0 * masks).T
  sv = jnp.stack([scores, values])
  out = _sc_switch(query, km3, sv)
  return out.reshape(B)
